# Initial kernel scaffold; baseline (speedup 1.0000x reference)
import jax
import jax.numpy as jnp
from jax.experimental import pallas as pl


def kernel(X, cluster_ids, V0, g0, b0, V1, g1, b1, V2, g2, b2, V3, g3, b3, V4, g4, b4):
    raise NotImplementedError("write your pallas kernel here")


# speedup vs baseline: 5.2251x; 5.2251x over previous
import functools

import jax
import jax.numpy as jnp
from jax import lax
from jax.experimental import pallas as pl
from jax.experimental.pallas import tpu as pltpu
from jax.experimental.pallas import tpu_sc as plsc

NCLU = 64
NFREQ = 10
NTOK = 32768
HID = 256
IN_DIM = 3 + 6 * NFREQ
BLK = 256
NBLK = NTOK // BLK
NPAIR = NBLK + NCLU
NW = 16
CHUNK = NTOK // NW
NROW = CHUNK // 128

def _route_body(ids_hbm, x_hbm, xs_hbm, inv_hbm, hist_hbm, offs_hbm,
                ids_v, hist2d_v, hist_v, histall_v, cursor_v, pos2d_v,
                xrows_v, offs_v, sem):
    wid = lax.axis_index("s")
    base = wid * CHUNK
    zeros16 = jnp.zeros((16,), jnp.int32)
    lane = lax.iota(jnp.int32, 16)

    pltpu.sync_copy(ids_hbm.at[pl.ds(base, CHUNK)], ids_v)
    for v in range(NCLU * 16 // 16):
        hist2d_v[pl.ds(v * 16, 16)] = zeros16

    def hbody(j, carry):
        idx = lane * 128 + j
        cid = plsc.load_gather(ids_v, [idx])
        slot = cid * 16 + lane
        cnt = plsc.load_gather(hist2d_v, [slot])
        plsc.store_scatter(hist2d_v, [slot], cnt + 1)
        return carry

    lax.fori_loop(0, 128, hbody, 0)

    for cv in range(NCLU // 16):
        acc = zeros16
        cbase = (cv * 16 + lane) * 16
        for l in range(16):
            acc = acc + plsc.load_gather(hist2d_v, [cbase + l])
        hist_v[pl.ds(cv * 16, 16)] = acc
    pltpu.sync_copy(hist_v, hist_hbm.at[wid])
    plsc.subcore_barrier()

    pltpu.sync_copy(hist_hbm, histall_v)
    carry = jnp.int32(0)
    basevecs = []
    for v in range(NCLU // 16):
        sl = pl.ds(v * 16, 16)
        tot = zeros16
        part = zeros16
        for w in range(NW):
            row = histall_v[w, sl]
            tot = tot + row
            wmask = jnp.full((16,), w, jnp.int32) < wid
            part = part + jnp.where(wmask, row, zeros16)
        excl = plsc.cumsum(tot) - tot
        offv = excl + carry
        offs_v[sl] = offv
        basevecs.append(offv + part)
        carry = carry + jnp.sum(tot)
    offs_v[pl.ds(NCLU, 16)] = jnp.full((16,), NTOK, jnp.int32)

    for c in range(NCLU):
        own = hist2d_v[pl.ds(c * 16, 16)]
        excl = plsc.cumsum(own) - own
        cursor_v[pl.ds(c * 16, 16)] = excl + basevecs[c // 16][c % 16]

    def pbody(j, carry):
        idx = lane * 128 + j
        cid = plsc.load_gather(ids_v, [idx])
        slot = cid * 16 + lane
        p = plsc.load_gather(cursor_v, [slot])
        plsc.store_scatter(cursor_v, [slot], p + 1)
        plsc.store_scatter(pos2d_v, [lane, jnp.full((16,), j, jnp.int32)], p)
        return carry

    lax.fori_loop(0, 128, pbody, 0)
    pltpu.sync_copy(pos2d_v, inv_hbm.at[pl.ds(wid * NROW, NROW)])

    pltpu.sync_copy(x_hbm.at[pl.ds(base, CHUNK)], xrows_v)
    for k in range(NROW):
        pltpu.async_copy(xrows_v.at[pl.ds(k * 128, 128)],
                         xs_hbm.at[pos2d_v.at[k]], sem).wait()

    @pl.when(wid == 0)
    def _():
        pltpu.sync_copy(offs_v, offs_hbm)


def _pairs_body(offs_ref, pb_ref, pc_ref, prs_ref, pre_ref):
    def init(i, carry):
        pb_ref[i] = NBLK - 1
        pc_ref[i] = NCLU - 1
        prs_ref[i] = 0
        pre_ref[i] = 0
        return carry

    lax.fori_loop(0, NPAIR, init, 0)

    def cbody(c, p):
        s = offs_ref[c]
        e = offs_ref[c + 1]

        def nonempty(p0):
            b1 = (e - 1) // BLK

            def wbody(bp):
                b, q = bp
                pb_ref[q] = b
                pc_ref[q] = c
                prs_ref[q] = s
                pre_ref[q] = e
                return (b + 1, q + 1)

            return lax.while_loop(lambda bp: bp[0] <= b1, wbody,
                                  (s // BLK, p0))[1]

        return lax.cond(e > s, nonempty, lambda q: q, p)

    lax.fori_loop(0, NCLU, cbody, jnp.int32(0))


_pairs = pl.pallas_call(
    _pairs_body,
    in_specs=[pl.BlockSpec(memory_space=pltpu.SMEM)],
    out_specs=[pl.BlockSpec(memory_space=pltpu.SMEM)] * 4,
    out_shape=[jax.ShapeDtypeStruct((NPAIR,), jnp.int32)] * 4,
)


def _mlp_body(pb_s, pc_s, prs_s, pre_s,
              x_ref, fr_ref, v0_ref, v1_ref, v2_ref, v3_ref,
              v4_ref, g0_ref, b0_ref, g1_ref, b1_ref, g2_ref, b2_ref,
              g3_ref, b3_ref, g4_ref, b4_ref, out_ref):
    i = pl.program_id(0)
    rs = prs_s[i]
    re = pre_s[i]
    blk = pb_s[i]

    @pl.when(re > rs)
    def _():
        f32 = jnp.float32
        dnt = (((1,), (1,)), ((), ()))

        x = x_ref[...]
        x3 = x[:, :3]
        xt = jnp.concatenate([x3] * NFREQ, axis=1)
        xb = xt * fr_ref[...]
        enc = jnp.concatenate([x3, jnp.sin(xb), jnp.cos(xb)], axis=1)

        def wn(v, gt):
            n2 = jnp.sum(v * v, axis=1, keepdims=True)
            return v * (gt * lax.rsqrt(n2))

        h = enc
        for v_ref, g_ref, b_ref in ((v0_ref, g0_ref, b0_ref),
                                    (v1_ref, g1_ref, b1_ref),
                                    (v2_ref, g2_ref, b2_ref),
                                    (v3_ref, g3_ref, b3_ref)):
            w = wn(v_ref[0], g_ref[0])
            h = lax.dot_general(h, w, dnt, preferred_element_type=f32)
            h = jnp.maximum(h + b_ref[0], 0.0)

        w4 = wn(v4_ref[0], g4_ref[0])
        y = jnp.tanh(lax.dot_general(h, w4, dnt, preferred_element_type=f32)
                     + b4_ref[0])

        rows = blk * BLK + lax.broadcasted_iota(jnp.int32, (BLK, 1), 0)
        m = (rows >= rs) & (rows < re)
        out_ref[...] = jnp.where(m, y, out_ref[...])


def _pb_map(i, pb, pc, rs, re):
    return (pb[i], 0)


def _pc_map(i, pb, pc, rs, re):
    return (pc[i], 0, 0)


_mlp = pl.pallas_call(
    _mlp_body,
    grid_spec=pltpu.PrefetchScalarGridSpec(
        num_scalar_prefetch=4,
        grid=(NPAIR,),
        in_specs=[
            pl.BlockSpec((BLK, 16), _pb_map),
            pl.BlockSpec((1, 3 * NFREQ), lambda i, *_: (0, 0)),
            pl.BlockSpec((1, HID, IN_DIM), _pc_map),
            pl.BlockSpec((1, HID, HID), _pc_map),
            pl.BlockSpec((1, HID, HID), _pc_map),
            pl.BlockSpec((1, HID, HID), _pc_map),
            pl.BlockSpec((1, 16, HID), _pc_map),
            pl.BlockSpec((1, HID, 1), _pc_map),
            pl.BlockSpec((1, 1, HID), _pc_map),
            pl.BlockSpec((1, HID, 1), _pc_map),
            pl.BlockSpec((1, 1, HID), _pc_map),
            pl.BlockSpec((1, HID, 1), _pc_map),
            pl.BlockSpec((1, 1, HID), _pc_map),
            pl.BlockSpec((1, HID, 1), _pc_map),
            pl.BlockSpec((1, 1, HID), _pc_map),
            pl.BlockSpec((1, 16, 1), _pc_map),
            pl.BlockSpec((1, 1, 16), _pc_map),
        ],
        out_specs=pl.BlockSpec((BLK, 16), _pb_map),
    ),
    out_shape=jax.ShapeDtypeStruct((NTOK, 16), jnp.float32),
    compiler_params=pltpu.CompilerParams(
        dimension_semantics=("arbitrary",)),
)


def _unsort_body(ys_hbm, inv_hbm, out_hbm, idx_v, rows_v, sem):
    wid = lax.axis_index("s")
    for k in range(NROW):
        r = wid * NROW + k
        pltpu.sync_copy(inv_hbm.at[r], idx_v)
        pltpu.async_copy(ys_hbm.at[idx_v], rows_v, sem).wait()
        pltpu.sync_copy(rows_v, out_hbm.at[pl.ds(r * 128, 128)])


@functools.lru_cache(maxsize=None)
def _sc_kernels():
    mesh = plsc.VectorSubcoreMesh(core_axis_name="c", subcore_axis_name="s",
                                  num_cores=1, num_subcores=NW)
    sc_params = pltpu.CompilerParams(needs_layout_passes=False,
                                     use_tc_tiling_on_sc=False)
    route = pl.kernel(
        _route_body,
        mesh=mesh,
        compiler_params=sc_params,
        out_type=(
            jax.ShapeDtypeStruct((NTOK, 16), jnp.float32),
            jax.ShapeDtypeStruct((NTOK // 128, 128), jnp.int32),
            jax.ShapeDtypeStruct((NW, NCLU), jnp.int32),
            jax.ShapeDtypeStruct((NCLU + 16,), jnp.int32),
        ),
        scratch_types=[
            pltpu.VMEM((CHUNK,), jnp.int32),
            pltpu.VMEM((NCLU * 16,), jnp.int32),
            pltpu.VMEM((NCLU,), jnp.int32),
            pltpu.VMEM((NW, NCLU), jnp.int32),
            pltpu.VMEM((NCLU * 16,), jnp.int32),
            pltpu.VMEM((NROW, 128), jnp.int32),
            pltpu.VMEM((CHUNK, 16), jnp.float32),
            pltpu.VMEM((NCLU + 16,), jnp.int32),
            pltpu.SemaphoreType.DMA,
        ],
    )
    unsort = pl.kernel(
        _unsort_body,
        mesh=mesh,
        compiler_params=sc_params,
        out_type=jax.ShapeDtypeStruct((NTOK, 16), jnp.float32),
        scratch_types=[
            pltpu.VMEM((128,), jnp.int32),
            pltpu.VMEM((128, 16), jnp.float32),
            pltpu.SemaphoreType.DMA,
        ],
    )
    return route, unsort


@jax.jit
def kernel(X, cluster_ids, V0, g0, b0, V1, g1, b1, V2, g2, b2, V3, g3, b3,
           V4, g4, b4):
    ids = cluster_ids.astype(jnp.int32)
    route, unsort = _sc_kernels()
    xp = jnp.pad(X, ((0, 0), (0, 13)))
    xs, inv, _hist, offs = route(ids, xp)
    pb, pc, prs, pre = _pairs(offs)

    rest = V0[:, :, 3:].reshape(NCLU, HID, NFREQ, 2, 3)
    vsin = rest[:, :, :, 0, :].reshape(NCLU, HID, 3 * NFREQ)
    vcos = rest[:, :, :, 1, :].reshape(NCLU, HID, 3 * NFREQ)
    v0p = jnp.concatenate([V0[:, :, :3], vsin, vcos], axis=2)
    freqs = 2.0 ** jnp.arange(NFREQ, dtype=jnp.float32)
    frow = jnp.kron(freqs, jnp.ones((3,), jnp.float32))[None]

    v4p = jnp.concatenate(
        [V4, jnp.zeros((NCLU, 13, HID), jnp.float32)
             .at[:, :, 0].set(1.0)], axis=1)
    g4p = jnp.pad(g4, ((0, 0), (0, 13)))
    b4p = jnp.pad(b4, ((0, 0), (0, 13)))

    ys = _mlp(pb, pc, prs, pre, xs, frow, v0p, V1, V2, V3, v4p,
              g0.reshape(NCLU, HID, 1), b0.reshape(NCLU, 1, HID),
              g1.reshape(NCLU, HID, 1), b1.reshape(NCLU, 1, HID),
              g2.reshape(NCLU, HID, 1), b2.reshape(NCLU, 1, HID),
              g3.reshape(NCLU, HID, 1), b3.reshape(NCLU, 1, HID),
              g4p.reshape(NCLU, 16, 1), b4p.reshape(NCLU, 1, 16))
    return unsort(ys, inv)[:, :3]


# speedup vs baseline: 6.1966x; 1.1859x over previous
import functools

import jax
import jax.numpy as jnp
from jax import lax
from jax.experimental import pallas as pl
from jax.experimental.pallas import tpu as pltpu
from jax.experimental.pallas import tpu_sc as plsc

NCLU = 64
NFREQ = 10
NTOK = 32768
HID = 256
IN_DIM = 3 + 6 * NFREQ
BLK = 256
NBLK = NTOK // BLK
NPAIR = NBLK + NCLU
NW = 16
CHUNK = NTOK // NW
NROW = CHUNK // 128

def _route_body(ids_hbm, x_hbm, xs_hbm, inv_hbm, hist_hbm, offs_hbm,
                ids_v, hist2d_v, hist_v, histall_v, cursor_v, pos2d_v,
                xrows_v, offs_v, sem):
    wid = lax.axis_index("s")
    base = wid * CHUNK
    zeros16 = jnp.zeros((16,), jnp.int32)
    lane = lax.iota(jnp.int32, 16)

    pltpu.sync_copy(ids_hbm.at[pl.ds(base, CHUNK)], ids_v)
    for v in range(NCLU * 16 // 16):
        hist2d_v[pl.ds(v * 16, 16)] = zeros16

    def hbody(j, carry):
        idx = lane * 128 + j
        cid = plsc.load_gather(ids_v, [idx])
        slot = cid * 16 + lane
        cnt = plsc.load_gather(hist2d_v, [slot])
        plsc.store_scatter(hist2d_v, [slot], cnt + 1)
        return carry

    lax.fori_loop(0, 128, hbody, 0)

    for cv in range(NCLU // 16):
        acc = zeros16
        cbase = (cv * 16 + lane) * 16
        for l in range(16):
            acc = acc + plsc.load_gather(hist2d_v, [cbase + l])
        hist_v[pl.ds(cv * 16, 16)] = acc
    pltpu.sync_copy(hist_v, hist_hbm.at[wid])
    plsc.subcore_barrier()

    pltpu.sync_copy(hist_hbm, histall_v)
    carry = jnp.int32(0)
    basevecs = []
    for v in range(NCLU // 16):
        sl = pl.ds(v * 16, 16)
        tot = zeros16
        part = zeros16
        for w in range(NW):
            row = histall_v[w, sl]
            tot = tot + row
            wmask = jnp.full((16,), w, jnp.int32) < wid
            part = part + jnp.where(wmask, row, zeros16)
        excl = plsc.cumsum(tot) - tot
        offv = excl + carry
        offs_v[sl] = offv
        basevecs.append(offv + part)
        carry = carry + jnp.sum(tot)
    offs_v[pl.ds(NCLU, 16)] = jnp.full((16,), NTOK, jnp.int32)

    for c in range(NCLU):
        own = hist2d_v[pl.ds(c * 16, 16)]
        excl = plsc.cumsum(own) - own
        cursor_v[pl.ds(c * 16, 16)] = excl + basevecs[c // 16][c % 16]

    def pbody(j, carry):
        idx = lane * 128 + j
        cid = plsc.load_gather(ids_v, [idx])
        slot = cid * 16 + lane
        p = plsc.load_gather(cursor_v, [slot])
        plsc.store_scatter(cursor_v, [slot], p + 1)
        plsc.store_scatter(pos2d_v, [lane, jnp.full((16,), j, jnp.int32)], p)
        return carry

    lax.fori_loop(0, 128, pbody, 0)
    pltpu.sync_copy(pos2d_v, inv_hbm.at[pl.ds(wid * NROW, NROW)])

    pltpu.sync_copy(x_hbm.at[pl.ds(base, CHUNK)], xrows_v)
    for k in range(NROW):
        pltpu.async_copy(xrows_v.at[pl.ds(k * 128, 128)],
                         xs_hbm.at[pos2d_v.at[k]], sem).wait()

    @pl.when(wid == 0)
    def _():
        pltpu.sync_copy(offs_v, offs_hbm)


def _pairs_body(offs_ref, pb_ref, pc_ref, prs_ref, pre_ref):
    def init(i, carry):
        pb_ref[i] = NBLK - 1
        pc_ref[i] = NCLU - 1
        prs_ref[i] = 0
        pre_ref[i] = 0
        return carry

    lax.fori_loop(0, NPAIR, init, 0)

    def cbody(c, p):
        s = offs_ref[c]
        e = offs_ref[c + 1]

        def nonempty(p0):
            b1 = (e - 1) // BLK

            def wbody(bp):
                b, q = bp
                pb_ref[q] = b
                pc_ref[q] = c
                prs_ref[q] = s
                pre_ref[q] = e
                return (b + 1, q + 1)

            return lax.while_loop(lambda bp: bp[0] <= b1, wbody,
                                  (s // BLK, p0))[1]

        return lax.cond(e > s, nonempty, lambda q: q, p)

    lax.fori_loop(0, NCLU, cbody, jnp.int32(0))


_pairs = pl.pallas_call(
    _pairs_body,
    in_specs=[pl.BlockSpec(memory_space=pltpu.SMEM)],
    out_specs=[pl.BlockSpec(memory_space=pltpu.SMEM)] * 4,
    out_shape=[jax.ShapeDtypeStruct((NPAIR,), jnp.int32)] * 4,
)


def _mlp_body(pb_s, pc_s, prs_s, pre_s,
              x_ref, fr_ref, vx_ref, vs_ref, vc_ref, v1_ref, v2_ref, v3_ref,
              v4_ref, g0_ref, b0_ref, g1_ref, b1_ref, g2_ref, b2_ref,
              g3_ref, b3_ref, g4_ref, b4_ref, out_ref,
              wx_sc, ws_sc, wc_sc, w1_sc, w2_sc, w3_sc, w4_sc):
    i = pl.program_id(0)
    rs = prs_s[i]
    re = pre_s[i]
    blk = pb_s[i]
    changed = (i == 0) | (pc_s[i] != pc_s[jnp.maximum(i - 1, 0)])
    f32 = jnp.float32
    dnt = (((1,), (1,)), ((), ()))

    @pl.when((re > rs) & changed)
    def _():
        def kd2(v):
            return jnp.sum(v * v, axis=1, keepdims=True)

        vx = vx_ref[0]
        vs = vs_ref[0]
        vc = vc_ref[0]
        s0 = g0_ref[0] * lax.rsqrt(kd2(vx) + kd2(vs) + kd2(vc))
        wx_sc[...] = vx * s0
        ws_sc[...] = vs * s0
        wc_sc[...] = vc * s0
        for v_ref, g_ref, w_sc in ((v1_ref, g1_ref, w1_sc),
                                   (v2_ref, g2_ref, w2_sc),
                                   (v3_ref, g3_ref, w3_sc)):
            v = v_ref[0]
            w_sc[...] = v * (g_ref[0] * lax.rsqrt(kd2(v)))
        v4 = v4_ref[0]
        w4_sc[...] = v4 * (g4_ref[0] * lax.rsqrt(kd2(v4)))

    @pl.when(re > rs)
    def _():
        x = x_ref[...]
        x3 = x[:, :3]
        xt = jnp.concatenate([x3] * NFREQ, axis=1)
        xb = xt * fr_ref[...]
        h = (lax.dot_general(x, wx_sc[...], dnt, preferred_element_type=f32)
             + lax.dot_general(jnp.sin(xb), ws_sc[...], dnt,
                               preferred_element_type=f32)
             + lax.dot_general(jnp.cos(xb), wc_sc[...], dnt,
                               preferred_element_type=f32))
        h = jnp.maximum(h + b0_ref[0], 0.0)
        for w_sc, b_ref in ((w1_sc, b1_ref), (w2_sc, b2_ref),
                            (w3_sc, b3_ref)):
            h = lax.dot_general(h, w_sc[...], dnt, preferred_element_type=f32)
            h = jnp.maximum(h + b_ref[0], 0.0)
        y = jnp.tanh(lax.dot_general(h, w4_sc[...], dnt,
                                     preferred_element_type=f32)
                     + b4_ref[0])

        rows = blk * BLK + lax.broadcasted_iota(jnp.int32, (BLK, 1), 0)
        m = (rows >= rs) & (rows < re)
        out_ref[...] = jnp.where(m, y, out_ref[...])


def _pb_map(i, pb, pc, rs, re):
    return (pb[i], 0)


def _pc_map(i, pb, pc, rs, re):
    return (pc[i], 0, 0)


_mlp = pl.pallas_call(
    _mlp_body,
    grid_spec=pltpu.PrefetchScalarGridSpec(
        num_scalar_prefetch=4,
        grid=(NPAIR,),
        in_specs=[
            pl.BlockSpec((BLK, 16), _pb_map),
            pl.BlockSpec((1, 3 * NFREQ), lambda i, *_: (0, 0)),
            pl.BlockSpec((1, HID, 16), _pc_map),
            pl.BlockSpec((1, HID, 3 * NFREQ), _pc_map),
            pl.BlockSpec((1, HID, 3 * NFREQ), _pc_map),
            pl.BlockSpec((1, HID, HID), _pc_map),
            pl.BlockSpec((1, HID, HID), _pc_map),
            pl.BlockSpec((1, HID, HID), _pc_map),
            pl.BlockSpec((1, 16, HID), _pc_map),
            pl.BlockSpec((1, HID, 1), _pc_map),
            pl.BlockSpec((1, 1, HID), _pc_map),
            pl.BlockSpec((1, HID, 1), _pc_map),
            pl.BlockSpec((1, 1, HID), _pc_map),
            pl.BlockSpec((1, HID, 1), _pc_map),
            pl.BlockSpec((1, 1, HID), _pc_map),
            pl.BlockSpec((1, HID, 1), _pc_map),
            pl.BlockSpec((1, 1, HID), _pc_map),
            pl.BlockSpec((1, 16, 1), _pc_map),
            pl.BlockSpec((1, 1, 16), _pc_map),
        ],
        out_specs=pl.BlockSpec((BLK, 16), _pb_map),
        scratch_shapes=[
            pltpu.VMEM((HID, 16), jnp.float32),
            pltpu.VMEM((HID, 3 * NFREQ), jnp.float32),
            pltpu.VMEM((HID, 3 * NFREQ), jnp.float32),
            pltpu.VMEM((HID, HID), jnp.float32),
            pltpu.VMEM((HID, HID), jnp.float32),
            pltpu.VMEM((HID, HID), jnp.float32),
            pltpu.VMEM((16, HID), jnp.float32),
        ],
    ),
    out_shape=jax.ShapeDtypeStruct((NTOK, 16), jnp.float32),
    compiler_params=pltpu.CompilerParams(
        dimension_semantics=("arbitrary",)),
)


def _unsort_body(ys_hbm, inv_hbm, out_hbm, idx_v, rows_v, sem):
    wid = lax.axis_index("s")
    for k in range(NROW):
        r = wid * NROW + k
        pltpu.sync_copy(inv_hbm.at[r], idx_v)
        pltpu.async_copy(ys_hbm.at[idx_v], rows_v, sem).wait()
        pltpu.sync_copy(rows_v, out_hbm.at[pl.ds(r * 128, 128)])


@functools.lru_cache(maxsize=None)
def _sc_kernels():
    mesh = plsc.VectorSubcoreMesh(core_axis_name="c", subcore_axis_name="s",
                                  num_cores=1, num_subcores=NW)
    sc_params = pltpu.CompilerParams(needs_layout_passes=False,
                                     use_tc_tiling_on_sc=False)
    route = pl.kernel(
        _route_body,
        mesh=mesh,
        compiler_params=sc_params,
        out_type=(
            jax.ShapeDtypeStruct((NTOK, 16), jnp.float32),
            jax.ShapeDtypeStruct((NTOK // 128, 128), jnp.int32),
            jax.ShapeDtypeStruct((NW, NCLU), jnp.int32),
            jax.ShapeDtypeStruct((NCLU + 16,), jnp.int32),
        ),
        scratch_types=[
            pltpu.VMEM((CHUNK,), jnp.int32),
            pltpu.VMEM((NCLU * 16,), jnp.int32),
            pltpu.VMEM((NCLU,), jnp.int32),
            pltpu.VMEM((NW, NCLU), jnp.int32),
            pltpu.VMEM((NCLU * 16,), jnp.int32),
            pltpu.VMEM((NROW, 128), jnp.int32),
            pltpu.VMEM((CHUNK, 16), jnp.float32),
            pltpu.VMEM((NCLU + 16,), jnp.int32),
            pltpu.SemaphoreType.DMA,
        ],
    )
    unsort = pl.kernel(
        _unsort_body,
        mesh=mesh,
        compiler_params=sc_params,
        out_type=jax.ShapeDtypeStruct((NTOK, 16), jnp.float32),
        scratch_types=[
            pltpu.VMEM((128,), jnp.int32),
            pltpu.VMEM((128, 16), jnp.float32),
            pltpu.SemaphoreType.DMA,
        ],
    )
    return route, unsort


@jax.jit
def kernel(X, cluster_ids, V0, g0, b0, V1, g1, b1, V2, g2, b2, V3, g3, b3,
           V4, g4, b4):
    ids = cluster_ids.astype(jnp.int32)
    route, unsort = _sc_kernels()
    xp = jnp.pad(X, ((0, 0), (0, 13)))
    xs, inv, _hist, offs = route(ids, xp)
    pb, pc, prs, pre = _pairs(offs)

    rest = V0[:, :, 3:].reshape(NCLU, HID, NFREQ, 2, 3)
    vsin = rest[:, :, :, 0, :].reshape(NCLU, HID, 3 * NFREQ)
    vcos = rest[:, :, :, 1, :].reshape(NCLU, HID, 3 * NFREQ)
    vx = jnp.pad(V0[:, :, :3], ((0, 0), (0, 0), (0, 13)))
    freqs = 2.0 ** jnp.arange(NFREQ, dtype=jnp.float32)
    frow = jnp.kron(freqs, jnp.ones((3,), jnp.float32))[None]

    v4p = jnp.concatenate(
        [V4, jnp.zeros((NCLU, 13, HID), jnp.float32)
             .at[:, :, 0].set(1.0)], axis=1)
    g4p = jnp.pad(g4, ((0, 0), (0, 13)))
    b4p = jnp.pad(b4, ((0, 0), (0, 13)))

    ys = _mlp(pb, pc, prs, pre, xs, frow, vx, vsin, vcos, V1, V2, V3, v4p,
              g0.reshape(NCLU, HID, 1), b0.reshape(NCLU, 1, HID),
              g1.reshape(NCLU, HID, 1), b1.reshape(NCLU, 1, HID),
              g2.reshape(NCLU, HID, 1), b2.reshape(NCLU, 1, HID),
              g3.reshape(NCLU, HID, 1), b3.reshape(NCLU, 1, HID),
              g4p.reshape(NCLU, 16, 1), b4p.reshape(NCLU, 1, 16))
    return unsort(ys, inv)[:, :3]


# speedup vs baseline: 6.2092x; 1.0020x over previous
import functools

import jax
import jax.numpy as jnp
from jax import lax
from jax.experimental import pallas as pl
from jax.experimental.pallas import tpu as pltpu
from jax.experimental.pallas import tpu_sc as plsc

NCLU = 64
NFREQ = 10
NTOK = 32768
HID = 256
IN_DIM = 3 + 6 * NFREQ
BLK = 256
NBLK = NTOK // BLK
NPAIR = NBLK + NCLU
NW = 16
CHUNK = NTOK // NW
NROW = CHUNK // 128

def _route_body(ids_hbm, x_hbm, xs_hbm, inv_hbm, hist_hbm, offs_hbm,
                ids_v, hist2d_v, hist_v, histall_v, cursor_v, pos2d_v,
                xrows_v, offs_v, sem):
    wid = lax.axis_index("s")
    base = wid * CHUNK
    zeros16 = jnp.zeros((16,), jnp.int32)
    lane = lax.iota(jnp.int32, 16)

    pltpu.sync_copy(ids_hbm.at[pl.ds(base, CHUNK)], ids_v)
    for v in range(NCLU * 16 // 16):
        hist2d_v[pl.ds(v * 16, 16)] = zeros16

    def hbody(j, carry):
        idx = lane * 128 + j
        cid = plsc.load_gather(ids_v, [idx])
        slot = cid * 16 + lane
        cnt = plsc.load_gather(hist2d_v, [slot])
        plsc.store_scatter(hist2d_v, [slot], cnt + 1)
        return carry

    lax.fori_loop(0, 128, hbody, 0)

    for cv in range(NCLU // 16):
        acc = zeros16
        cbase = (cv * 16 + lane) * 16
        for l in range(16):
            acc = acc + plsc.load_gather(hist2d_v, [cbase + l])
        hist_v[pl.ds(cv * 16, 16)] = acc
    pltpu.sync_copy(hist_v, hist_hbm.at[wid])
    plsc.subcore_barrier()

    pltpu.sync_copy(hist_hbm, histall_v)
    carry = jnp.int32(0)
    basevecs = []
    for v in range(NCLU // 16):
        sl = pl.ds(v * 16, 16)
        tot = zeros16
        part = zeros16
        for w in range(NW):
            row = histall_v[w, sl]
            tot = tot + row
            wmask = jnp.full((16,), w, jnp.int32) < wid
            part = part + jnp.where(wmask, row, zeros16)
        excl = plsc.cumsum(tot) - tot
        offv = excl + carry
        offs_v[sl] = offv
        basevecs.append(offv + part)
        carry = carry + jnp.sum(tot)
    offs_v[pl.ds(NCLU, 16)] = jnp.full((16,), NTOK, jnp.int32)

    for c in range(NCLU):
        own = hist2d_v[pl.ds(c * 16, 16)]
        excl = plsc.cumsum(own) - own
        cursor_v[pl.ds(c * 16, 16)] = excl + basevecs[c // 16][c % 16]

    def pbody(j, carry):
        idx = lane * 128 + j
        cid = plsc.load_gather(ids_v, [idx])
        slot = cid * 16 + lane
        p = plsc.load_gather(cursor_v, [slot])
        plsc.store_scatter(cursor_v, [slot], p + 1)
        plsc.store_scatter(pos2d_v, [lane, jnp.full((16,), j, jnp.int32)], p)
        return carry

    lax.fori_loop(0, 128, pbody, 0)
    pltpu.sync_copy(pos2d_v, inv_hbm.at[pl.ds(wid * NROW, NROW)])

    pltpu.sync_copy(x_hbm.at[pl.ds(base, CHUNK)], xrows_v)
    for k in range(NROW):
        pltpu.async_copy(xrows_v.at[pl.ds(k * 128, 128)],
                         xs_hbm.at[pos2d_v.at[k]], sem).wait()

    @pl.when(wid == 0)
    def _():
        pltpu.sync_copy(offs_v, offs_hbm)


def _pairs_body(offs_ref, pb_ref, pc_ref, prs_ref, pre_ref):
    def init(i, carry):
        pb_ref[i] = NBLK - 1
        pc_ref[i] = NCLU - 1
        prs_ref[i] = 0
        pre_ref[i] = 0
        return carry

    lax.fori_loop(0, NPAIR, init, 0)

    def cbody(c, p):
        s = offs_ref[c]
        e = offs_ref[c + 1]

        def nonempty(p0):
            b1 = (e - 1) // BLK

            def wbody(bp):
                b, q = bp
                pb_ref[q] = b
                pc_ref[q] = c
                prs_ref[q] = s
                pre_ref[q] = e
                return (b + 1, q + 1)

            return lax.while_loop(lambda bp: bp[0] <= b1, wbody,
                                  (s // BLK, p0))[1]

        return lax.cond(e > s, nonempty, lambda q: q, p)

    lax.fori_loop(0, NCLU, cbody, jnp.int32(0))


_pairs = pl.pallas_call(
    _pairs_body,
    in_specs=[pl.BlockSpec(memory_space=pltpu.SMEM)],
    out_specs=[pl.BlockSpec(memory_space=pltpu.SMEM)] * 4,
    out_shape=[jax.ShapeDtypeStruct((NPAIR,), jnp.int32)] * 4,
)


def _mlp_body(pb_s, pc_s, prs_s, pre_s,
              x_ref, fr_ref, vx_ref, vs_ref, vc_ref, v1_ref, v2_ref, v3_ref,
              v4_ref, g0_ref, b0_ref, g1_ref, b1_ref, g2_ref, b2_ref,
              g3_ref, b3_ref, g4_ref, b4_ref, out_ref,
              wx_sc, ws_sc, wc_sc, w1_sc, w2_sc, w3_sc, w4_sc):
    i = pl.program_id(0)
    rs = prs_s[i]
    re = pre_s[i]
    blk = pb_s[i]
    changed = (i == 0) | (pc_s[i] != pc_s[jnp.maximum(i - 1, 0)])
    f32 = jnp.float32
    dns = (((1,), (0,)), ((), ()))

    @pl.when((re > rs) & changed)
    def _():
        def kd2(v):
            return jnp.sum(v * v, axis=1, keepdims=True)

        vx = vx_ref[0]
        vs = vs_ref[0]
        vc = vc_ref[0]
        s0 = g0_ref[0] * lax.rsqrt(kd2(vx) + kd2(vs) + kd2(vc))
        wx_sc[...] = (vx * s0).T
        ws_sc[...] = (vs * s0).T
        wc_sc[...] = (vc * s0).T
        for v_ref, g_ref, w_sc in ((v1_ref, g1_ref, w1_sc),
                                   (v2_ref, g2_ref, w2_sc),
                                   (v3_ref, g3_ref, w3_sc)):
            v = v_ref[0]
            w_sc[...] = (v * (g_ref[0] * lax.rsqrt(kd2(v)))).T
        v4 = v4_ref[0]
        w4_sc[...] = (v4 * (g4_ref[0] * lax.rsqrt(kd2(v4)))).T

    @pl.when(re > rs)
    def _():
        x = x_ref[...]
        x3 = x[:, :3]
        xt = jnp.concatenate([x3] * NFREQ, axis=1)
        xb = xt * fr_ref[...]
        h = (lax.dot_general(x, wx_sc[...], dns, preferred_element_type=f32)
             + lax.dot_general(jnp.sin(xb), ws_sc[...], dns,
                               preferred_element_type=f32)
             + lax.dot_general(jnp.cos(xb), wc_sc[...], dns,
                               preferred_element_type=f32))
        h = jnp.maximum(h + b0_ref[0], 0.0)
        for w_sc, b_ref in ((w1_sc, b1_ref), (w2_sc, b2_ref),
                            (w3_sc, b3_ref)):
            h = lax.dot_general(h, w_sc[...], dns, preferred_element_type=f32)
            h = jnp.maximum(h + b_ref[0], 0.0)
        y = jnp.tanh(lax.dot_general(h, w4_sc[...], dns,
                                     preferred_element_type=f32)
                     + b4_ref[0])

        rows = blk * BLK + lax.broadcasted_iota(jnp.int32, (BLK, 1), 0)
        m = (rows >= rs) & (rows < re)
        out_ref[...] = jnp.where(m, y, out_ref[...])


def _pb_map(i, pb, pc, rs, re):
    return (pb[i], 0)


def _pc_map(i, pb, pc, rs, re):
    return (pc[i], 0, 0)


_mlp = pl.pallas_call(
    _mlp_body,
    grid_spec=pltpu.PrefetchScalarGridSpec(
        num_scalar_prefetch=4,
        grid=(NPAIR,),
        in_specs=[
            pl.BlockSpec((BLK, 16), _pb_map),
            pl.BlockSpec((1, 3 * NFREQ), lambda i, *_: (0, 0)),
            pl.BlockSpec((1, HID, 16), _pc_map),
            pl.BlockSpec((1, HID, 3 * NFREQ), _pc_map),
            pl.BlockSpec((1, HID, 3 * NFREQ), _pc_map),
            pl.BlockSpec((1, HID, HID), _pc_map),
            pl.BlockSpec((1, HID, HID), _pc_map),
            pl.BlockSpec((1, HID, HID), _pc_map),
            pl.BlockSpec((1, 16, HID), _pc_map),
            pl.BlockSpec((1, HID, 1), _pc_map),
            pl.BlockSpec((1, 1, HID), _pc_map),
            pl.BlockSpec((1, HID, 1), _pc_map),
            pl.BlockSpec((1, 1, HID), _pc_map),
            pl.BlockSpec((1, HID, 1), _pc_map),
            pl.BlockSpec((1, 1, HID), _pc_map),
            pl.BlockSpec((1, HID, 1), _pc_map),
            pl.BlockSpec((1, 1, HID), _pc_map),
            pl.BlockSpec((1, 16, 1), _pc_map),
            pl.BlockSpec((1, 1, 16), _pc_map),
        ],
        out_specs=pl.BlockSpec((BLK, 16), _pb_map),
        scratch_shapes=[
            pltpu.VMEM((16, HID), jnp.float32),
            pltpu.VMEM((3 * NFREQ, HID), jnp.float32),
            pltpu.VMEM((3 * NFREQ, HID), jnp.float32),
            pltpu.VMEM((HID, HID), jnp.float32),
            pltpu.VMEM((HID, HID), jnp.float32),
            pltpu.VMEM((HID, HID), jnp.float32),
            pltpu.VMEM((HID, 16), jnp.float32),
        ],
    ),
    out_shape=jax.ShapeDtypeStruct((NTOK, 16), jnp.float32),
    compiler_params=pltpu.CompilerParams(
        dimension_semantics=("arbitrary",)),
)


def _unsort_body(ys_hbm, inv_hbm, out_hbm, idx_v, rows_v, sem):
    wid = lax.axis_index("s")
    for k in range(NROW):
        r = wid * NROW + k
        pltpu.sync_copy(inv_hbm.at[r], idx_v)
        pltpu.async_copy(ys_hbm.at[idx_v], rows_v, sem).wait()
        pltpu.sync_copy(rows_v, out_hbm.at[pl.ds(r * 128, 128)])


@functools.lru_cache(maxsize=None)
def _sc_kernels():
    mesh = plsc.VectorSubcoreMesh(core_axis_name="c", subcore_axis_name="s",
                                  num_cores=1, num_subcores=NW)
    sc_params = pltpu.CompilerParams(needs_layout_passes=False,
                                     use_tc_tiling_on_sc=False)
    route = pl.kernel(
        _route_body,
        mesh=mesh,
        compiler_params=sc_params,
        out_type=(
            jax.ShapeDtypeStruct((NTOK, 16), jnp.float32),
            jax.ShapeDtypeStruct((NTOK // 128, 128), jnp.int32),
            jax.ShapeDtypeStruct((NW, NCLU), jnp.int32),
            jax.ShapeDtypeStruct((NCLU + 16,), jnp.int32),
        ),
        scratch_types=[
            pltpu.VMEM((CHUNK,), jnp.int32),
            pltpu.VMEM((NCLU * 16,), jnp.int32),
            pltpu.VMEM((NCLU,), jnp.int32),
            pltpu.VMEM((NW, NCLU), jnp.int32),
            pltpu.VMEM((NCLU * 16,), jnp.int32),
            pltpu.VMEM((NROW, 128), jnp.int32),
            pltpu.VMEM((CHUNK, 16), jnp.float32),
            pltpu.VMEM((NCLU + 16,), jnp.int32),
            pltpu.SemaphoreType.DMA,
        ],
    )
    unsort = pl.kernel(
        _unsort_body,
        mesh=mesh,
        compiler_params=sc_params,
        out_type=jax.ShapeDtypeStruct((NTOK, 16), jnp.float32),
        scratch_types=[
            pltpu.VMEM((128,), jnp.int32),
            pltpu.VMEM((128, 16), jnp.float32),
            pltpu.SemaphoreType.DMA,
        ],
    )
    return route, unsort


@jax.jit
def kernel(X, cluster_ids, V0, g0, b0, V1, g1, b1, V2, g2, b2, V3, g3, b3,
           V4, g4, b4):
    ids = cluster_ids.astype(jnp.int32)
    route, unsort = _sc_kernels()
    xp = jnp.pad(X, ((0, 0), (0, 13)))
    xs, inv, _hist, offs = route(ids, xp)
    pb, pc, prs, pre = _pairs(offs)

    rest = V0[:, :, 3:].reshape(NCLU, HID, NFREQ, 2, 3)
    vsin = rest[:, :, :, 0, :].reshape(NCLU, HID, 3 * NFREQ)
    vcos = rest[:, :, :, 1, :].reshape(NCLU, HID, 3 * NFREQ)
    vx = jnp.pad(V0[:, :, :3], ((0, 0), (0, 0), (0, 13)))
    freqs = 2.0 ** jnp.arange(NFREQ, dtype=jnp.float32)
    frow = jnp.kron(freqs, jnp.ones((3,), jnp.float32))[None]

    v4p = jnp.concatenate(
        [V4, jnp.zeros((NCLU, 13, HID), jnp.float32)
             .at[:, :, 0].set(1.0)], axis=1)
    g4p = jnp.pad(g4, ((0, 0), (0, 13)))
    b4p = jnp.pad(b4, ((0, 0), (0, 13)))

    ys = _mlp(pb, pc, prs, pre, xs, frow, vx, vsin, vcos, V1, V2, V3, v4p,
              g0.reshape(NCLU, HID, 1), b0.reshape(NCLU, 1, HID),
              g1.reshape(NCLU, HID, 1), b1.reshape(NCLU, 1, HID),
              g2.reshape(NCLU, HID, 1), b2.reshape(NCLU, 1, HID),
              g3.reshape(NCLU, HID, 1), b3.reshape(NCLU, 1, HID),
              g4p.reshape(NCLU, 16, 1), b4p.reshape(NCLU, 1, 16))
    return unsort(ys, inv)[:, :3]


# speedup vs baseline: 6.4962x; 1.0462x over previous
import functools

import jax
import jax.numpy as jnp
from jax import lax
from jax.experimental import pallas as pl
from jax.experimental.pallas import tpu as pltpu
from jax.experimental.pallas import tpu_sc as plsc

NCLU = 64
NFREQ = 10
NTOK = 32768
HID = 256
IN_DIM = 3 + 6 * NFREQ
BLK = 512
NBLK = NTOK // BLK
NPAIR = NBLK + NCLU
NW = 16
CHUNK = NTOK // NW
NROW = CHUNK // 128

def _route_body(ids_hbm, x_hbm, xs_hbm, inv_hbm, hist_hbm, offs_hbm,
                ids_v, hist2d_v, hist_v, histall_v, cursor_v, pos2d_v,
                xrows_v, offs_v, sem):
    wid = lax.axis_index("s")
    base = wid * CHUNK
    zeros16 = jnp.zeros((16,), jnp.int32)
    lane = lax.iota(jnp.int32, 16)

    pltpu.sync_copy(ids_hbm.at[pl.ds(base, CHUNK)], ids_v)
    for v in range(NCLU * 16 // 16):
        hist2d_v[pl.ds(v * 16, 16)] = zeros16

    def hbody(j, carry):
        idx = lane * 128 + j
        cid = plsc.load_gather(ids_v, [idx])
        slot = cid * 16 + lane
        cnt = plsc.load_gather(hist2d_v, [slot])
        plsc.store_scatter(hist2d_v, [slot], cnt + 1)
        return carry

    lax.fori_loop(0, 128, hbody, 0)

    for cv in range(NCLU // 16):
        acc = zeros16
        cbase = (cv * 16 + lane) * 16
        for l in range(16):
            acc = acc + plsc.load_gather(hist2d_v, [cbase + l])
        hist_v[pl.ds(cv * 16, 16)] = acc
    pltpu.sync_copy(hist_v, hist_hbm.at[wid])
    plsc.subcore_barrier()

    pltpu.sync_copy(hist_hbm, histall_v)
    carry = jnp.int32(0)
    basevecs = []
    for v in range(NCLU // 16):
        sl = pl.ds(v * 16, 16)
        tot = zeros16
        part = zeros16
        for w in range(NW):
            row = histall_v[w, sl]
            tot = tot + row
            wmask = jnp.full((16,), w, jnp.int32) < wid
            part = part + jnp.where(wmask, row, zeros16)
        excl = plsc.cumsum(tot) - tot
        offv = excl + carry
        offs_v[sl] = offv
        basevecs.append(offv + part)
        carry = carry + jnp.sum(tot)
    offs_v[pl.ds(NCLU, 16)] = jnp.full((16,), NTOK, jnp.int32)

    for c in range(NCLU):
        own = hist2d_v[pl.ds(c * 16, 16)]
        excl = plsc.cumsum(own) - own
        cursor_v[pl.ds(c * 16, 16)] = excl + basevecs[c // 16][c % 16]

    def pbody(j, carry):
        idx = lane * 128 + j
        cid = plsc.load_gather(ids_v, [idx])
        slot = cid * 16 + lane
        p = plsc.load_gather(cursor_v, [slot])
        plsc.store_scatter(cursor_v, [slot], p + 1)
        plsc.store_scatter(pos2d_v, [lane, jnp.full((16,), j, jnp.int32)], p)
        return carry

    lax.fori_loop(0, 128, pbody, 0)
    pltpu.sync_copy(pos2d_v, inv_hbm.at[pl.ds(wid * NROW, NROW)])

    pltpu.sync_copy(x_hbm.at[pl.ds(base, CHUNK)], xrows_v)
    for k in range(NROW):
        pltpu.async_copy(xrows_v.at[pl.ds(k * 128, 128)],
                         xs_hbm.at[pos2d_v.at[k]], sem).wait()

    @pl.when(wid == 0)
    def _():
        pltpu.sync_copy(offs_v, offs_hbm)


def _pairs_body(offs_ref, pb_ref, pc_ref, prs_ref, pre_ref):
    def init(i, carry):
        pb_ref[i] = NBLK - 1
        pc_ref[i] = NCLU - 1
        prs_ref[i] = 0
        pre_ref[i] = 0
        return carry

    lax.fori_loop(0, NPAIR, init, 0)

    def cbody(c, p):
        s = offs_ref[c]
        e = offs_ref[c + 1]

        def nonempty(p0):
            b1 = (e - 1) // BLK

            def wbody(bp):
                b, q = bp
                pb_ref[q] = b
                pc_ref[q] = c
                prs_ref[q] = s
                pre_ref[q] = e
                return (b + 1, q + 1)

            return lax.while_loop(lambda bp: bp[0] <= b1, wbody,
                                  (s // BLK, p0))[1]

        return lax.cond(e > s, nonempty, lambda q: q, p)

    lax.fori_loop(0, NCLU, cbody, jnp.int32(0))


_pairs = pl.pallas_call(
    _pairs_body,
    in_specs=[pl.BlockSpec(memory_space=pltpu.SMEM)],
    out_specs=[pl.BlockSpec(memory_space=pltpu.SMEM)] * 4,
    out_shape=[jax.ShapeDtypeStruct((NPAIR,), jnp.int32)] * 4,
)


def _mlp_body(pb_s, pc_s, prs_s, pre_s,
              x_ref, fr_ref, vx_ref, vs_ref, vc_ref, v1_ref, v2_ref, v3_ref,
              v4_ref, g0_ref, b0_ref, g1_ref, b1_ref, g2_ref, b2_ref,
              g3_ref, b3_ref, g4_ref, b4_ref, out_ref,
              wx_sc, ws_sc, wc_sc, w1_sc, w2_sc, w3_sc, w4_sc):
    i = pl.program_id(0)
    rs = prs_s[i]
    re = pre_s[i]
    blk = pb_s[i]
    changed = (i == 0) | (pc_s[i] != pc_s[jnp.maximum(i - 1, 0)])
    f32 = jnp.float32
    dns = (((1,), (0,)), ((), ()))

    @pl.when((re > rs) & changed)
    def _():
        def kd2(v):
            return jnp.sum(v * v, axis=1, keepdims=True)

        vx = vx_ref[0]
        vs = vs_ref[0]
        vc = vc_ref[0]
        s0 = g0_ref[0] * lax.rsqrt(kd2(vx) + kd2(vs) + kd2(vc))
        wx_sc[...] = (vx * s0).T
        ws_sc[...] = (vs * s0).T
        wc_sc[...] = (vc * s0).T
        for v_ref, g_ref, w_sc in ((v1_ref, g1_ref, w1_sc),
                                   (v2_ref, g2_ref, w2_sc),
                                   (v3_ref, g3_ref, w3_sc)):
            v = v_ref[0]
            w_sc[...] = (v * (g_ref[0] * lax.rsqrt(kd2(v)))).T
        v4 = v4_ref[0]
        w4_sc[...] = (v4 * (g4_ref[0] * lax.rsqrt(kd2(v4)))).T

    @pl.when(re > rs)
    def _():
        x = x_ref[...]
        x3 = x[:, :3]
        xt = jnp.concatenate([x3] * NFREQ, axis=1)
        xb = xt * fr_ref[...]
        h = (lax.dot_general(x, wx_sc[...], dns, preferred_element_type=f32)
             + lax.dot_general(jnp.sin(xb), ws_sc[...], dns,
                               preferred_element_type=f32)
             + lax.dot_general(jnp.cos(xb), wc_sc[...], dns,
                               preferred_element_type=f32))
        h = jnp.maximum(h + b0_ref[0], 0.0)
        for w_sc, b_ref in ((w1_sc, b1_ref), (w2_sc, b2_ref),
                            (w3_sc, b3_ref)):
            h = lax.dot_general(h, w_sc[...], dns, preferred_element_type=f32)
            h = jnp.maximum(h + b_ref[0], 0.0)
        y = jnp.tanh(lax.dot_general(h, w4_sc[...], dns,
                                     preferred_element_type=f32)
                     + b4_ref[0])

        rows = blk * BLK + lax.broadcasted_iota(jnp.int32, (BLK, 1), 0)
        m = (rows >= rs) & (rows < re)
        out_ref[...] = jnp.where(m, y, out_ref[...])


def _pb_map(i, pb, pc, rs, re):
    return (pb[i], 0)


def _pc_map(i, pb, pc, rs, re):
    return (pc[i], 0, 0)


_mlp = pl.pallas_call(
    _mlp_body,
    grid_spec=pltpu.PrefetchScalarGridSpec(
        num_scalar_prefetch=4,
        grid=(NPAIR,),
        in_specs=[
            pl.BlockSpec((BLK, 16), _pb_map),
            pl.BlockSpec((1, 3 * NFREQ), lambda i, *_: (0, 0)),
            pl.BlockSpec((1, HID, 16), _pc_map),
            pl.BlockSpec((1, HID, 3 * NFREQ), _pc_map),
            pl.BlockSpec((1, HID, 3 * NFREQ), _pc_map),
            pl.BlockSpec((1, HID, HID), _pc_map),
            pl.BlockSpec((1, HID, HID), _pc_map),
            pl.BlockSpec((1, HID, HID), _pc_map),
            pl.BlockSpec((1, 16, HID), _pc_map),
            pl.BlockSpec((1, HID, 1), _pc_map),
            pl.BlockSpec((1, 1, HID), _pc_map),
            pl.BlockSpec((1, HID, 1), _pc_map),
            pl.BlockSpec((1, 1, HID), _pc_map),
            pl.BlockSpec((1, HID, 1), _pc_map),
            pl.BlockSpec((1, 1, HID), _pc_map),
            pl.BlockSpec((1, HID, 1), _pc_map),
            pl.BlockSpec((1, 1, HID), _pc_map),
            pl.BlockSpec((1, 16, 1), _pc_map),
            pl.BlockSpec((1, 1, 16), _pc_map),
        ],
        out_specs=pl.BlockSpec((BLK, 16), _pb_map),
        scratch_shapes=[
            pltpu.VMEM((16, HID), jnp.float32),
            pltpu.VMEM((3 * NFREQ, HID), jnp.float32),
            pltpu.VMEM((3 * NFREQ, HID), jnp.float32),
            pltpu.VMEM((HID, HID), jnp.float32),
            pltpu.VMEM((HID, HID), jnp.float32),
            pltpu.VMEM((HID, HID), jnp.float32),
            pltpu.VMEM((HID, 16), jnp.float32),
        ],
    ),
    out_shape=jax.ShapeDtypeStruct((NTOK, 16), jnp.float32),
    compiler_params=pltpu.CompilerParams(
        dimension_semantics=("arbitrary",)),
)


def _unsort_body(ys_hbm, inv_hbm, out_hbm, idx_v, rows_v, sem):
    wid = lax.axis_index("s")
    for k in range(NROW):
        r = wid * NROW + k
        pltpu.sync_copy(inv_hbm.at[r], idx_v)
        pltpu.async_copy(ys_hbm.at[idx_v], rows_v, sem).wait()
        pltpu.sync_copy(rows_v, out_hbm.at[pl.ds(r * 128, 128)])


@functools.lru_cache(maxsize=None)
def _sc_kernels():
    mesh = plsc.VectorSubcoreMesh(core_axis_name="c", subcore_axis_name="s",
                                  num_cores=1, num_subcores=NW)
    sc_params = pltpu.CompilerParams(needs_layout_passes=False,
                                     use_tc_tiling_on_sc=False)
    route = pl.kernel(
        _route_body,
        mesh=mesh,
        compiler_params=sc_params,
        out_type=(
            jax.ShapeDtypeStruct((NTOK, 16), jnp.float32),
            jax.ShapeDtypeStruct((NTOK // 128, 128), jnp.int32),
            jax.ShapeDtypeStruct((NW, NCLU), jnp.int32),
            jax.ShapeDtypeStruct((NCLU + 16,), jnp.int32),
        ),
        scratch_types=[
            pltpu.VMEM((CHUNK,), jnp.int32),
            pltpu.VMEM((NCLU * 16,), jnp.int32),
            pltpu.VMEM((NCLU,), jnp.int32),
            pltpu.VMEM((NW, NCLU), jnp.int32),
            pltpu.VMEM((NCLU * 16,), jnp.int32),
            pltpu.VMEM((NROW, 128), jnp.int32),
            pltpu.VMEM((CHUNK, 16), jnp.float32),
            pltpu.VMEM((NCLU + 16,), jnp.int32),
            pltpu.SemaphoreType.DMA,
        ],
    )
    unsort = pl.kernel(
        _unsort_body,
        mesh=mesh,
        compiler_params=sc_params,
        out_type=jax.ShapeDtypeStruct((NTOK, 16), jnp.float32),
        scratch_types=[
            pltpu.VMEM((128,), jnp.int32),
            pltpu.VMEM((128, 16), jnp.float32),
            pltpu.SemaphoreType.DMA,
        ],
    )
    return route, unsort


@jax.jit
def kernel(X, cluster_ids, V0, g0, b0, V1, g1, b1, V2, g2, b2, V3, g3, b3,
           V4, g4, b4):
    ids = cluster_ids.astype(jnp.int32)
    route, unsort = _sc_kernels()
    xp = jnp.pad(X, ((0, 0), (0, 13)))
    xs, inv, _hist, offs = route(ids, xp)
    pb, pc, prs, pre = _pairs(offs)

    rest = V0[:, :, 3:].reshape(NCLU, HID, NFREQ, 2, 3)
    vsin = rest[:, :, :, 0, :].reshape(NCLU, HID, 3 * NFREQ)
    vcos = rest[:, :, :, 1, :].reshape(NCLU, HID, 3 * NFREQ)
    vx = jnp.pad(V0[:, :, :3], ((0, 0), (0, 0), (0, 13)))
    freqs = 2.0 ** jnp.arange(NFREQ, dtype=jnp.float32)
    frow = jnp.kron(freqs, jnp.ones((3,), jnp.float32))[None]

    v4p = jnp.concatenate(
        [V4, jnp.zeros((NCLU, 13, HID), jnp.float32)
             .at[:, :, 0].set(1.0)], axis=1)
    g4p = jnp.pad(g4, ((0, 0), (0, 13)))
    b4p = jnp.pad(b4, ((0, 0), (0, 13)))

    ys = _mlp(pb, pc, prs, pre, xs, frow, vx, vsin, vcos, V1, V2, V3, v4p,
              g0.reshape(NCLU, HID, 1), b0.reshape(NCLU, 1, HID),
              g1.reshape(NCLU, HID, 1), b1.reshape(NCLU, 1, HID),
              g2.reshape(NCLU, HID, 1), b2.reshape(NCLU, 1, HID),
              g3.reshape(NCLU, HID, 1), b3.reshape(NCLU, 1, HID),
              g4p.reshape(NCLU, 16, 1), b4p.reshape(NCLU, 1, 16))
    return unsort(ys, inv)[:, :3]
